# Initial kernel scaffold; baseline (speedup 1.0000x reference)
import jax
import jax.numpy as jnp
from jax.experimental import pallas as pl


def kernel(pos, x, lig_coords, W1, b1, gamma, beta, run_mean, run_var, W2, b2):
    raise NotImplementedError("write your pallas kernel here")


# speedup vs baseline: 1.8227x; 1.8227x over previous
import functools
import jax
import jax.numpy as jnp
from jax.experimental import pallas as pl
from jax.experimental.pallas import tpu as pltpu

_K = 10
_BIG = 3.0e38


def _knn_pool_body(lig_ref, posx_ref, posy_ref, posz_ref, x_ref,
                   w1_ref, b1_ref, w2_ref, b2_ref, out_ref, d2_ref,
                   vmask_ref, xbuf_ref, sem):
    npad = d2_ref.shape[1]
    bi = pl.program_id(0)
    nsc = 14
    sc = npad // nsc
    iota_c = jax.lax.broadcasted_iota(jnp.int32, (64, sc), 1)

    lx = lig_ref[0, :, 0:1]
    ly = lig_ref[0, :, 1:2]
    lz = lig_ref[0, :, 2:3]

    def acc_min(acc, rminc, idxc):
        minacc, idxacc = acc
        take = rminc < minacc
        return (jnp.where(take, rminc, minacc),
                jnp.where(take, idxc, idxacc))

    def rowmin(d2c, ii):
        rminc = jnp.min(d2c, axis=1, keepdims=True)
        cand = jnp.where(d2c == rminc, ii, npad)
        idxc = jnp.min(cand, axis=1, keepdims=True)
        return rminc, idxc

    acc0 = (jnp.full((64, 1), _BIG, jnp.float32),
            jnp.full((64, 1), npad, jnp.int32))

    def first_chunk(c, acc):
        s = c * sc
        dx = lx - posx_ref[0, 0:1, pl.ds(s, sc)]
        dy = ly - posy_ref[0, 0:1, pl.ds(s, sc)]
        dz = lz - posz_ref[0, 0:1, pl.ds(s, sc)]
        d2c = dx * dx + dy * dy + dz * dz
        d2_ref[:, pl.ds(s, sc)] = d2c
        vmask_ref[:, pl.ds(s, sc)] = jnp.zeros((1, sc), jnp.float32)
        return acc_min(acc, *rowmin(d2c, iota_c + s))

    _, idx = jax.lax.fori_loop(0, nsc, first_chunk, acc0)

    def kiter(_, prev_idx):
        def chunk_step(c, acc):
            s = c * sc
            ii = iota_c + s
            chosen = ii == prev_idx
            d2c = jnp.where(chosen, _BIG, d2_ref[:, pl.ds(s, sc)])
            d2_ref[:, pl.ds(s, sc)] = d2c
            hit = jnp.max(chosen.astype(jnp.float32), axis=0, keepdims=True)
            vmask_ref[:, pl.ds(s, sc)] = jnp.maximum(
                vmask_ref[:, pl.ds(s, sc)], hit)
            return acc_min(acc, *rowmin(d2c, ii))

        _, nidx = jax.lax.fori_loop(0, nsc, chunk_step, acc0)
        return nidx

    idx = jax.lax.fori_loop(0, _K - 1, kiter, idx)

    nbuf, ch, d = xbuf_ref.shape
    nchunks = npad // ch
    iota_p = jax.lax.broadcasted_iota(jnp.int32, (64, ch), 1)

    def _copy(c, slot):
        return pltpu.make_async_copy(
            x_ref.at[bi, pl.ds(c * ch, ch), :], xbuf_ref.at[slot],
            sem.at[slot])

    _copy(0, 0).start()

    def pool_step(c, carry):
        acc, cnt = carry
        slot = jax.lax.rem(c, nbuf)

        @pl.when(c + 1 < nchunks)
        def _():
            _copy(c + 1, jax.lax.rem(c + 1, nbuf)).start()

        s = c * ch
        last_hit = jnp.max((iota_p + s == idx).astype(jnp.float32),
                           axis=0, keepdims=True)
        mchunk = jnp.maximum(vmask_ref[:, pl.ds(s, ch)], last_hit)
        _copy(c, slot).wait()
        acc = acc + jax.lax.dot_general(
            mchunk, xbuf_ref[slot], (((1,), (0,)), ((), ())),
            preferred_element_type=jnp.float32)
        return acc, cnt + jnp.sum(mchunk)

    pooled, count = jax.lax.fori_loop(
        0, nchunks, pool_step,
        (jnp.zeros((1, d), dtype=jnp.float32), jnp.float32(0.0)))
    mean = pooled * (1.0 / count)

    h = jax.lax.dot_general(
        mean, w1_ref[...], (((1,), (1,)), ((), ())),
        preferred_element_type=jnp.float32) + b1_ref[...]
    h = h * jax.nn.sigmoid(h)
    out = jax.lax.dot_general(
        h, w2_ref[...], (((1,), (0,)), ((), ())),
        preferred_element_type=jnp.float32) + b2_ref[...]
    out_ref[0] = out


@functools.partial(jax.jit, static_argnames=())
def kernel(pos, x, lig_coords, W1, b1, gamma, beta, run_mean, run_var, W2, b2):
    B, N, D = x.shape
    L = lig_coords.shape[1]
    OUT = W2.shape[0]
    NPAD = 50176
    CH = NPAD // 8
    assert N <= NPAD

    posT = jnp.transpose(pos, (0, 2, 1))
    posT = jnp.pad(posT, ((0, 0), (0, 0), (0, NPAD - N)),
                   constant_values=1.0e4)
    posx = posT[:, 0:1, :]
    posy = posT[:, 1:2, :]
    posz = posT[:, 2:3, :]
    xp = jnp.pad(x, ((0, 0), (0, NPAD - N), (0, 0)))

    scale = gamma * jax.lax.rsqrt(run_var + 1e-5)
    W1f = W1 * scale[:, None]
    b1f = ((b1 - run_mean) * scale + beta)[None, :]

    OPAD = ((OUT + 127) // 128) * 128
    W2T = jnp.pad(W2.T, ((0, 0), (0, OPAD - OUT)))
    b2p = jnp.pad(b2, (0, OPAD - OUT))[None, :]

    out = pl.pallas_call(
        _knn_pool_body,
        grid=(B,),
        in_specs=[
            pl.BlockSpec((1, L, 3), lambda b: (b, 0, 0)),
            pl.BlockSpec((1, 1, NPAD), lambda b: (b, 0, 0)),
            pl.BlockSpec((1, 1, NPAD), lambda b: (b, 0, 0)),
            pl.BlockSpec((1, 1, NPAD), lambda b: (b, 0, 0)),
            pl.BlockSpec(memory_space=pl.ANY),
            pl.BlockSpec((D, D), lambda b: (0, 0)),
            pl.BlockSpec((1, D), lambda b: (0, 0)),
            pl.BlockSpec((D, OPAD), lambda b: (0, 0)),
            pl.BlockSpec((1, OPAD), lambda b: (0, 0)),
        ],
        out_specs=pl.BlockSpec((1, 1, OPAD), lambda b: (b, 0, 0)),
        out_shape=jax.ShapeDtypeStruct((B, 1, OPAD), jnp.float32),
        scratch_shapes=[pltpu.VMEM((64, NPAD), jnp.float32),
                        pltpu.VMEM((1, NPAD), jnp.float32),
                        pltpu.VMEM((2, CH, D), jnp.float32),
                        pltpu.SemaphoreType.DMA((2,))],
    )(lig_coords, posx, posy, posz, xp, W1f, b1f, W2T, b2p)
    return out[:, 0, :OUT]
